# baseline (speedup 1.0000x reference)
import functools

import jax
import jax.numpy as jnp
from jax import lax
from jax.experimental import pallas as pl
from jax.experimental.pallas import tpu as pltpu
from jax.experimental.pallas import tpu_sc as plsc

_N = 320000
_D = 128

_K_SC = 64000
_NW = 32
_RPW = _K_SC // _NW
_C = 200
_NCH = _RPW // _C
_UNROLL = 8

_BLK = 6400
_TC_BASE = _K_SC // _BLK
_TC_GRID = (_N - _K_SC) // _BLK


def _acc_rows(buf, it, accs):
    new = list(accs)
    r = it * _UNROLL
    for u in range(_UNROLL):
        for j in range(8):
            new[j] = new[j] + buf[r + u, pl.ds(16 * j, 16)]
    return tuple(new)


def _phase(hbm, base, buf0, buf1, sem0, sem1, accvm):
    bufs = (buf0, buf1)
    sems = (sem0, sem1)
    pltpu.make_async_copy(hbm.at[pl.ds(base, _C)], buf0, sem0).start()
    pltpu.make_async_copy(hbm.at[pl.ds(base + _C, _C)], buf1, sem1).start()

    def outer(g, _):
        for b in range(2):
            k = g * 2 + b
            pltpu.make_async_copy(
                hbm.at[pl.ds(base + k * _C, _C)], bufs[b], sems[b]
            ).wait()
            zeros = tuple(jnp.zeros((16,), jnp.float32) for _ in range(8))
            accs = lax.fori_loop(
                0, _C // _UNROLL, functools.partial(_acc_rows, bufs[b]), zeros
            )

            @pl.when(k + 2 < _NCH)
            def _refill():
                pltpu.make_async_copy(
                    hbm.at[pl.ds(base + (k + 2) * _C, _C)], bufs[b], sems[b]
                ).start()

            for j in range(8):
                accvm[pl.ds(16 * j, 16)] = accvm[pl.ds(16 * j, 16)] + accs[j]
        return 0

    lax.fori_loop(0, _NCH // 2, outer, 0)


@functools.partial(
    pl.kernel,
    out_type=jax.ShapeDtypeStruct((_NW, _D), jnp.float32),
    mesh=plsc.VectorSubcoreMesh(core_axis_name="c", subcore_axis_name="s"),
    scratch_types=[
        pltpu.VMEM((_C, _D), jnp.float32),
        pltpu.VMEM((_C, _D), jnp.float32),
        pltpu.VMEM((_D,), jnp.float32),
        pltpu.SemaphoreType.DMA,
        pltpu.SemaphoreType.DMA,
    ],
)
def _sc_sum(a_hbm, b_hbm, out_hbm, buf0, buf1, accvm, sem0, sem1):
    wid = lax.axis_index("s") * 2 + lax.axis_index("c")
    base = wid * _RPW
    zero = jnp.zeros((16,), jnp.float32)
    for j in range(8):
        accvm[pl.ds(16 * j, 16)] = zero
    _phase(a_hbm, base, buf0, buf1, sem0, sem1, accvm)
    _phase(b_hbm, base, buf0, buf1, sem0, sem1, accvm)
    pltpu.sync_copy(accvm, out_hbm.at[wid])


def _tc_body(a_ref, b_ref, out_ref, acc_ref):
    step = pl.program_id(0)
    grid = pl.num_programs(0)
    a = a_ref[...].reshape(_BLK // 8, 8, _D)
    b = b_ref[...].reshape(_BLK // 8, 8, _D)
    partial = jnp.sum(a, axis=0) + jnp.sum(b, axis=0)

    @pl.when(step == 0)
    def _init():
        acc_ref[...] = partial

    @pl.when(step != 0)
    def _acc():
        acc_ref[...] += partial

    @pl.when(step == grid - 1)
    def _final():
        out_ref[...] = jnp.sum(
            acc_ref[...].reshape(1, 8, _D), axis=1
        )


def _tc_sum(attrs_0, attrs_1, base_blk, grid):
    return pl.pallas_call(
        _tc_body,
        grid=(grid,),
        in_specs=[
            pl.BlockSpec((_BLK, _D), lambda i: (i + base_blk, 0)),
            pl.BlockSpec((_BLK, _D), lambda i: (i + base_blk, 0)),
        ],
        out_specs=pl.BlockSpec((1, _D), lambda i: (0, 0)),
        out_shape=jax.ShapeDtypeStruct((1, _D), jnp.float32),
        scratch_shapes=[pltpu.VMEM((8, _D), jnp.float32)],
    )(attrs_0, attrs_1)


def kernel(attrs_0, attrs_1):
    out = _tc_sum(attrs_0, attrs_1, 0, _N // _BLK)
    return out.reshape(_D)


# speedup vs baseline: 1.0444x; 1.0444x over previous
import jax
import jax.numpy as jnp
from jax.experimental import pallas as pl
from jax.experimental.pallas import tpu as pltpu

_N = 320000
_D = 128
_BLK = 10000


def _sum_body(a_ref, b_ref, out_ref, acc_ref):
    step = pl.program_id(0)
    grid = pl.num_programs(0)
    a = a_ref[...].reshape(_BLK // 8, 8, _D)
    b = b_ref[...].reshape(_BLK // 8, 8, _D)
    partial = jnp.sum(a, axis=0) + jnp.sum(b, axis=0)

    @pl.when(step == 0)
    def _init():
        acc_ref[...] = partial

    @pl.when(step != 0)
    def _acc():
        acc_ref[...] += partial

    @pl.when(step == grid - 1)
    def _final():
        out_ref[...] = jnp.sum(acc_ref[...].reshape(1, 8, _D), axis=1)


def kernel(attrs_0, attrs_1):
    out = pl.pallas_call(
        _sum_body,
        grid=(_N // _BLK,),
        in_specs=[
            pl.BlockSpec((_BLK, _D), lambda i: (i, 0)),
            pl.BlockSpec((_BLK, _D), lambda i: (i, 0)),
        ],
        out_specs=pl.BlockSpec((1, _D), lambda i: (0, 0)),
        out_shape=jax.ShapeDtypeStruct((1, _D), jnp.float32),
        scratch_shapes=[pltpu.VMEM((8, _D), jnp.float32)],
    )(attrs_0, attrs_1)
    return out.reshape(_D)
